# Initial kernel scaffold; baseline (speedup 1.0000x reference)
import jax
import jax.numpy as jnp
from jax.experimental import pallas as pl


def kernel(x_user, x_item, W_in_user, b_in_user, W_in_item, b_in_item, W_ui, al_ui, ar_ui, b_ui, W_iu, al_iu, ar_iu, b_iu, W_out_user, b_out_user, W_out_item, b_out_item, edge_index_ui, edge_index_iu):
    raise NotImplementedError("write your pallas kernel here")


# speedup vs baseline: 7.5023x; 7.5023x over previous
import jax
import jax.numpy as jnp
from jax.experimental import pallas as pl

_N = 50000
_HID = 128
_OUT = 64
_HEADS = 4
_DH = _HID // _HEADS
_BLK = 1000
_GRID = _N // _BLK


def _node_body(x_ref, win_ref, bin_ref, wsrc_ref, wdst_ref, al_ref, ar_ref,
               fs_ref, el_ref, er_ref):
    x = x_ref[...]
    h = jnp.maximum(
        jnp.dot(x, win_ref[...], preferred_element_type=jnp.float32)
        + bin_ref[...], 0.0)
    fs = jnp.dot(h, wsrc_ref[...], preferred_element_type=jnp.float32)
    fs_ref[...] = fs
    el_ref[...] = jnp.dot(fs, al_ref[...], preferred_element_type=jnp.float32)
    fd = jnp.dot(h, wdst_ref[...], preferred_element_type=jnp.float32)
    er_ref[...] = jnp.dot(fd, ar_ref[...], preferred_element_type=jnp.float32)


def _out_body(feat_ref, bh_ref, wout_ref, bout_ref, o_ref):
    f = feat_ref[...] + bh_ref[...]
    o_ref[...] = (jnp.dot(f, wout_ref[...], preferred_element_type=jnp.float32)
                  + bout_ref[...])


def _attn_mat(a):
    m = jnp.zeros((_HID, _HEADS), a.dtype)
    for h in range(_HEADS):
        m = m.at[h * _DH:(h + 1) * _DH, h].set(a[h])
    return m


def _node_stage(x, w_in, b_in, w_src, w_dst, al_src, ar_dst):
    full = lambda shp: pl.BlockSpec(shp, lambda i: (0, 0))
    return pl.pallas_call(
        _node_body,
        grid=(_GRID,),
        in_specs=[
            pl.BlockSpec((_BLK, _HID), lambda i: (i, 0)),
            full((_HID, _HID)),
            full((1, _HID)),
            full((_HID, _HID)),
            full((_HID, _HID)),
            full((_HID, _HEADS)),
            full((_HID, _HEADS)),
        ],
        out_specs=[
            pl.BlockSpec((_BLK, _HID), lambda i: (i, 0)),
            pl.BlockSpec((_BLK, _HEADS), lambda i: (i, 0)),
            pl.BlockSpec((_BLK, _HEADS), lambda i: (i, 0)),
        ],
        out_shape=[
            jax.ShapeDtypeStruct((_N, _HID), jnp.float32),
            jax.ShapeDtypeStruct((_N, _HEADS), jnp.float32),
            jax.ShapeDtypeStruct((_N, _HEADS), jnp.float32),
        ],
    )(x, w_in, b_in.reshape(1, _HID), w_src, w_dst,
      _attn_mat(al_src), _attn_mat(ar_dst))


def _out_stage(feat, b_head, w_out, b_out):
    full = lambda shp: pl.BlockSpec(shp, lambda i: (0, 0))
    return pl.pallas_call(
        _out_body,
        grid=(_GRID,),
        in_specs=[
            pl.BlockSpec((_BLK, _HID), lambda i: (i, 0)),
            full((1, _HID)),
            full((_HID, _OUT)),
            full((1, _OUT)),
        ],
        out_specs=pl.BlockSpec((_BLK, _OUT), lambda i: (i, 0)),
        out_shape=jax.ShapeDtypeStruct((_N, _OUT), jnp.float32),
    )(feat, b_head.reshape(1, _HID), w_out, b_out.reshape(1, _OUT))


def _edge_softmax(fs_src, el_src, er_dst, src, dst, n_dst):
    e = jax.nn.leaky_relu(el_src[src] + er_dst[dst], negative_slope=0.2)
    emax = jax.ops.segment_max(e, dst, num_segments=n_dst)
    emax = jnp.where(jnp.isfinite(emax), emax, 0.0)
    ee = jnp.exp(e - emax[dst])
    den = jax.ops.segment_sum(ee, dst, num_segments=n_dst)
    alpha = ee / den[dst]
    msg = jnp.repeat(alpha, _DH, axis=1) * fs_src[src]
    return jax.ops.segment_sum(msg, dst, num_segments=n_dst)


def kernel(x_user, x_item, W_in_user, b_in_user, W_in_item, b_in_item,
           W_ui, al_ui, ar_ui, b_ui, W_iu, al_iu, ar_iu, b_iu,
           W_out_user, b_out_user, W_out_item, b_out_item,
           edge_index_ui, edge_index_iu):
    fs_user, el_user, er_user = _node_stage(
        x_user, W_in_user, b_in_user, W_ui, W_iu, al_ui, ar_iu)
    fs_item, el_item, er_item = _node_stage(
        x_item, W_in_item, b_in_item, W_iu, W_ui, al_iu, ar_ui)

    feat_item = _edge_softmax(fs_user, el_user, er_item,
                              edge_index_ui[0], edge_index_ui[1], _N)
    feat_user = _edge_softmax(fs_item, el_item, er_user,
                              edge_index_iu[0], edge_index_iu[1], _N)

    out_user = _out_stage(feat_user, b_iu, W_out_user, b_out_user)
    out_item = _out_stage(feat_item, b_ui, W_out_item, b_out_item)
    return (out_user, out_item)
